# Initial kernel scaffold; baseline (speedup 1.0000x reference)
import jax
import jax.numpy as jnp
from jax.experimental import pallas as pl


def kernel(nodes, edge_sources, edge_targets, rij, combine_sets, plane_wave, graph_indices, node_counts, params):
    raise NotImplementedError("write your pallas kernel here")


# speedup vs baseline: 2.0663x; 2.0663x over previous
import functools

import jax
import jax.numpy as jnp
from jax import lax
from jax.experimental import pallas as pl
from jax.experimental.pallas import tpu as pltpu
from jax.experimental.pallas import tpu_sc as plsc

_N_NODES = 10000
_N_EDGES = 320000
_D = 128
_NGRAPH = 64

_NC = 2
_NS = 16
_NW = _NC * _NS
_EPW = _N_EDGES // _NW
_C = 80
_NCHUNK = _EPW // _C

_RPT = 624
_RTAIL = _N_NODES - _NS * _RPT

_NODE_TILE = 400
_NODE_GRID = _N_NODES // _NODE_TILE
_EDGE_TILE = 512
_EDGE_GRID = _N_EDGES // _EDGE_TILE


def _sigmoid(x):
    return 1.0 / (1.0 + jnp.exp(-x))


def _elu(x):
    return jnp.where(x > 0, x, jnp.exp(x) - 1.0)


def _dot(a, b):
    return jnp.dot(a, b, preferred_element_type=jnp.float32)


@functools.lru_cache(maxsize=None)
def _sc_kernels():
    mesh = plsc.VectorSubcoreMesh(core_axis_name="c", subcore_axis_name="s",
                                  num_cores=_NC, num_subcores=_NS)

    @functools.partial(
        pl.kernel,
        out_type=(
            jax.ShapeDtypeStruct((_N_EDGES, _D), jnp.float32),
            jax.ShapeDtypeStruct((_N_EDGES, _D), jnp.float32),
        ),
        mesh=mesh,
        scratch_types=[
            pltpu.VMEM((_C,), jnp.int32),
            pltpu.VMEM((_C,), jnp.int32),
            pltpu.VMEM((_C, _D), jnp.float32),
            pltpu.VMEM((_C, _D), jnp.float32),
            pltpu.SemaphoreType.DMA,
            pltpu.SemaphoreType.DMA,
        ],
    )
    def sc_gather(x_hbm, src_hbm, dst_hbm, ni_hbm, nj_hbm,
                  sidx, didx, srows, drows, sem_s, sem_d):
        cid = lax.axis_index("c")
        sid = lax.axis_index("s")
        base = (cid * _NS + sid) * _EPW

        def body(j, carry):
            off = base + j * _C
            pltpu.sync_copy(src_hbm.at[pl.ds(off, _C)], sidx)
            pltpu.sync_copy(dst_hbm.at[pl.ds(off, _C)], didx)
            cp_s = pltpu.async_copy(x_hbm.at[sidx], srows, sem_s)
            cp_d = pltpu.async_copy(x_hbm.at[didx], drows, sem_d)
            cp_s.wait()
            cp_d.wait()
            pltpu.sync_copy(srows, ni_hbm.at[pl.ds(off, _C)])
            pltpu.sync_copy(drows, nj_hbm.at[pl.ds(off, _C)])
            return carry

        lax.fori_loop(0, _NCHUNK, body, 0)

    @functools.partial(
        pl.kernel,
        out_type=jax.ShapeDtypeStruct((_NC, _N_NODES, _D), jnp.float32),
        mesh=mesh,
        scratch_types=[
            pltpu.VMEM((_C,), jnp.int32),
            pltpu.VMEM((_C, _D), jnp.float32),
            pltpu.VMEM_SHARED((_N_NODES, _D), jnp.float32),
        ],
    )
    def sc_scatter(z_hbm, src_hbm, zeros_hbm, upd_hbm, idxb, zb, accum):
        cid = lax.axis_index("c")
        sid = lax.axis_index("s")

        pltpu.sync_copy(zeros_hbm.at[pl.ds(sid * _RPT, _RPT)],
                        accum.at[pl.ds(sid * _RPT, _RPT)])

        @pl.when(sid == 0)
        def _():
            pltpu.sync_copy(zeros_hbm.at[pl.ds(_NS * _RPT, _RTAIL)],
                            accum.at[pl.ds(_NS * _RPT, _RTAIL)])

        plsc.subcore_barrier()

        base = (cid * _NS + sid) * _EPW

        def body(j, carry):
            off = base + j * _C
            pltpu.sync_copy(src_hbm.at[pl.ds(off, _C)], idxb)
            pltpu.sync_copy(z_hbm.at[pl.ds(off, _C)], zb)
            pltpu.sync_copy(zb, accum.at[idxb], add=True)
            return carry

        lax.fori_loop(0, _NCHUNK, body, 0)

        plsc.subcore_barrier()

        pltpu.sync_copy(accum.at[pl.ds(sid * _RPT, _RPT)],
                        upd_hbm.at[cid, pl.ds(sid * _RPT, _RPT)])

        @pl.when(sid == 0)
        def _():
            pltpu.sync_copy(accum.at[pl.ds(_NS * _RPT, _RTAIL)],
                            upd_hbm.at[cid, pl.ds(_NS * _RPT, _RTAIL)])

    return sc_gather, sc_scatter


def _embed_body(n_ref, w_ref, o_ref):
    o_ref[...] = _sigmoid(_dot(n_ref[...], w_ref[...]))


def _embed(nodes, emb_w):
    return pl.pallas_call(
        _embed_body,
        grid=(_NODE_GRID,),
        in_specs=[
            pl.BlockSpec((_NODE_TILE, _D), lambda i: (i, 0)),
            pl.BlockSpec((_D, _D), lambda i: (0, 0)),
        ],
        out_specs=pl.BlockSpec((_NODE_TILE, _D), lambda i: (i, 0)),
        out_shape=jax.ShapeDtypeStruct((_N_NODES, _D), jnp.float32),
    )(nodes, emb_w)


def _edge_body(ni_ref, nj_ref, rij_ref, cs_ref, pw_ref,
               g1, g2, g3, m1, m2, m3, w1, w2, w2g, z_ref):
    ii = ni_ref[...]
    jj = nj_ref[...]
    inv_r = 1.0 / rij_ref[...]
    dd = (jj - ii) * inv_r
    gate = _sigmoid(_dot(ii, g1[...]) + _dot(jj, g2[...]) + _dot(dd, g3[...]))
    mlp = _elu(_dot(ii, m1[...]) + _dot(jj, m2[...]) + _dot(dd, m3[...]))
    pw = pw_ref[...]
    zc = _dot(cs_ref[...], w1[...]) + _dot(pw, w2[...]) * _sigmoid(_dot(pw, w2g[...]))
    z_ref[...] = gate * mlp * zc


def _edge(ni, nj, rij2, cs, pw, g1, g2, g3, m1, m2, m3, w1, w2, w2g):
    ew = lambda i: (i, 0)
    cw = lambda i: (0, 0)
    return pl.pallas_call(
        _edge_body,
        grid=(_EDGE_GRID,),
        in_specs=[
            pl.BlockSpec((_EDGE_TILE, _D), ew),
            pl.BlockSpec((_EDGE_TILE, _D), ew),
            pl.BlockSpec((_EDGE_TILE, 1), ew),
            pl.BlockSpec((_EDGE_TILE, 36), ew),
            pl.BlockSpec((_EDGE_TILE, 26), ew),
            pl.BlockSpec((_D, _D), cw),
            pl.BlockSpec((_D, _D), cw),
            pl.BlockSpec((_D, _D), cw),
            pl.BlockSpec((_D, _D), cw),
            pl.BlockSpec((_D, _D), cw),
            pl.BlockSpec((_D, _D), cw),
            pl.BlockSpec((36, _D), cw),
            pl.BlockSpec((26, _D), cw),
            pl.BlockSpec((26, _D), cw),
        ],
        out_specs=pl.BlockSpec((_EDGE_TILE, _D), ew),
        out_shape=jax.ShapeDtypeStruct((_N_EDGES, _D), jnp.float32),
    )(ni, nj, rij2, cs, pw, g1, g2, g3, m1, m2, m3, w1, w2, w2g)


def _node_body(x_ref, u_ref, q_ref, psi, p1, p2, xo_ref, qo_ref):
    xv = x_ref[...] + u_ref[0] + u_ref[1]
    xn = _elu(_dot(xv, psi[...]))
    zp = _elu(_dot(xn, p1[...]))
    xo_ref[...] = xn
    qo_ref[...] = q_ref[...] + zp * _dot(xn, p2[...])


def _node(x, upd, q, psi, p1, p2):
    nw = lambda i: (i, 0)
    cw = lambda i: (0, 0)
    return pl.pallas_call(
        _node_body,
        grid=(_NODE_GRID,),
        in_specs=[
            pl.BlockSpec((_NODE_TILE, _D), nw),
            pl.BlockSpec((_NC, _NODE_TILE, _D), lambda i: (0, i, 0)),
            pl.BlockSpec((_NODE_TILE, _D), nw),
            pl.BlockSpec((_D, _D), cw),
            pl.BlockSpec((_D, _D), cw),
            pl.BlockSpec((_D, _D), cw),
        ],
        out_specs=[
            pl.BlockSpec((_NODE_TILE, _D), nw),
            pl.BlockSpec((_NODE_TILE, _D), nw),
        ],
        out_shape=[
            jax.ShapeDtypeStruct((_N_NODES, _D), jnp.float32),
            jax.ShapeDtypeStruct((_N_NODES, _D), jnp.float32),
        ],
    )(x, upd, q, psi, p1, p2)


def _final_body(q_ref, gi_ref, w1, w2, w3, y_ref, pooled):
    i = pl.program_id(0)

    @pl.when(i == 0)
    def _():
        pooled[...] = jnp.zeros_like(pooled)

    gi = gi_ref[0, 0, :]
    gids = lax.broadcasted_iota(jnp.int32, (_NGRAPH, _NODE_TILE), 0)
    onehot = (gids == gi[None, :]).astype(jnp.float32)
    pooled[...] += _dot(onehot, q_ref[...])

    @pl.when(i == _NODE_GRID - 1)
    def _():
        y = _elu(_dot(pooled[...], w1[...]))
        y = _elu(_dot(y, w2[...]))
        y_ref[...] = _dot(y, w3[...])


def _final(q, gi3, w1, w2, w3):
    cw = lambda i: (0, 0)
    return pl.pallas_call(
        _final_body,
        grid=(_NODE_GRID,),
        in_specs=[
            pl.BlockSpec((_NODE_TILE, _D), lambda i: (i, 0)),
            pl.BlockSpec((1, 1, _NODE_TILE), lambda i: (i, 0, 0)),
            pl.BlockSpec((_D, 64), cw),
            pl.BlockSpec((64, 42), cw),
            pl.BlockSpec((42, 1), cw),
        ],
        out_specs=pl.BlockSpec((_NGRAPH, 1), cw),
        out_shape=jax.ShapeDtypeStruct((_NGRAPH, 1), jnp.float32),
        scratch_shapes=[pltpu.VMEM((_NGRAPH, _D), jnp.float32)],
    )(q, gi3, w1, w2, w3)


@jax.jit
def _run(nodes, src, dst, rij2, cs, pw, gi3, params):
    sc_gather, sc_scatter = _sc_kernels()
    zeros_nodes = jnp.zeros((_N_NODES, _D), jnp.float32)
    x = _embed(nodes, params["emb_W"])
    q = jnp.zeros((_N_NODES, _D), jnp.float32)
    for bp in params["blocks"]:
        g1, g2, g3 = jnp.split(bp["lin_gate"], 3, axis=0)
        m1, m2, m3 = jnp.split(bp["lin_mlp"], 3, axis=0)
        ni, nj = sc_gather(x, src, dst)
        z = _edge(ni, nj, rij2, cs, pw, g1, g2, g3, m1, m2, m3,
                  bp["lin1_vec"], bp["lin2_vec"], bp["lin2_vec_gate"])
        upd = sc_scatter(z, src, zeros_nodes)
        x, q = _node(x, upd, q, bp["psi_W"], bp["pool_W1"], bp["pool_W2"])
    return _final(q, gi3, params["lr"][0], params["lr"][1], params["lr"][2])


def kernel(nodes, edge_sources, edge_targets, rij, combine_sets, plane_wave,
           graph_indices, node_counts, params):
    del node_counts
    src = edge_sources.astype(jnp.int32)
    dst = edge_targets.astype(jnp.int32)
    rij2 = rij.reshape(_N_EDGES, 1)
    gi3 = graph_indices.astype(jnp.int32).reshape(_NODE_GRID, 1, _NODE_TILE)
    return _run(nodes, src, dst, rij2, combine_sets, plane_wave, gi3, params)


# speedup vs baseline: 2.5442x; 1.2313x over previous
import functools

import jax
import jax.numpy as jnp
from jax import lax
from jax.experimental import pallas as pl
from jax.experimental.pallas import tpu as pltpu
from jax.experimental.pallas import tpu_sc as plsc

_N_NODES = 10000
_N_EDGES = 320000
_D = 128
_NGRAPH = 64

_NC = 2
_NS = 16
_NW = _NC * _NS
_EPW = _N_EDGES // _NW
_C = 80
_NCHUNK = _EPW // _C

_RPT = 624
_RTAIL = _N_NODES - _NS * _RPT

_NODE_TILE = 400
_NODE_GRID = _N_NODES // _NODE_TILE
_EDGE_TILE = 512
_EDGE_GRID = _N_EDGES // _EDGE_TILE


def _sigmoid(x):
    return 1.0 / (1.0 + jnp.exp(-x))


def _elu(x):
    return jnp.where(x > 0, x, jnp.exp(x) - 1.0)


def _dot(a, b):
    return jnp.dot(a, b, preferred_element_type=jnp.float32)


_NBUF = 5
_NGRP = _NCHUNK // _NBUF

_NBUF_S = 4
_NGRP_S = (_NCHUNK - 1) // _NBUF_S


@functools.lru_cache(maxsize=None)
def _sc_kernels():
    mesh = plsc.VectorSubcoreMesh(core_axis_name="c", subcore_axis_name="s",
                                  num_cores=_NC, num_subcores=_NS)

    gather_scratch = (
        [pltpu.VMEM((_EPW,), jnp.int32)] * 2
        + [pltpu.VMEM((_C, _D), jnp.float32)] * (2 * _NBUF)
        + [pltpu.SemaphoreType.DMA] * (2 * _NBUF)
    )

    @functools.partial(
        pl.kernel,
        out_type=(
            jax.ShapeDtypeStruct((_N_EDGES, _D), jnp.float32),
            jax.ShapeDtypeStruct((_N_EDGES, _D), jnp.float32),
        ),
        mesh=mesh,
        scratch_types=gather_scratch,
    )
    def sc_gather(x_hbm, src_hbm, dst_hbm, ni_hbm, nj_hbm, *scr):
        sidx, didx = scr[0], scr[1]
        sbufs = scr[2:2 + _NBUF]
        dbufs = scr[2 + _NBUF:2 + 2 * _NBUF]
        gsems = scr[2 + 2 * _NBUF:2 + 3 * _NBUF]
        wsems = scr[2 + 3 * _NBUF:2 + 4 * _NBUF]

        cid = lax.axis_index("c")
        sid = lax.axis_index("s")
        wid = cid * _NS + sid
        base = wid * _EPW

        pltpu.sync_copy(src_hbm.at[pl.ds(base, _EPW)], sidx)
        pltpu.sync_copy(dst_hbm.at[pl.ds(base, _EPW)], didx)

        def s_at(j):
            return sidx.at[pl.ds(j * _C, _C)]

        def d_at(j):
            return didx.at[pl.ds(j * _C, _C)]

        for b in range(_NBUF):
            pltpu.async_copy(x_hbm.at[s_at(b)], sbufs[b], gsems[b])
            pltpu.async_copy(x_hbm.at[d_at(b)], dbufs[b], gsems[b])

        def group(g, carry):
            for b in range(_NBUF):
                j = g * _NBUF + b
                off = base + j * _C
                pltpu.make_async_copy(x_hbm.at[s_at(j)], sbufs[b],
                                      gsems[b]).wait()
                pltpu.make_async_copy(x_hbm.at[d_at(j)], dbufs[b],
                                      gsems[b]).wait()
                pltpu.async_copy(sbufs[b], ni_hbm.at[pl.ds(off, _C)], wsems[b])
                pltpu.async_copy(dbufs[b], nj_hbm.at[pl.ds(off, _C)], wsems[b])

            @pl.when(g < _NGRP - 1)
            def _():
                for b in range(_NBUF):
                    off = base + (g * _NBUF + b) * _C
                    jn = (g + 1) * _NBUF + b
                    pltpu.make_async_copy(sbufs[b], ni_hbm.at[pl.ds(off, _C)],
                                          wsems[b]).wait()
                    pltpu.make_async_copy(dbufs[b], nj_hbm.at[pl.ds(off, _C)],
                                          wsems[b]).wait()
                    pltpu.async_copy(x_hbm.at[s_at(jn)], sbufs[b], gsems[b])
                    pltpu.async_copy(x_hbm.at[d_at(jn)], dbufs[b], gsems[b])
            return carry

        lax.fori_loop(0, _NGRP, group, 0)

        for b in range(_NBUF):
            off = base + ((_NGRP - 1) * _NBUF + b) * _C
            pltpu.make_async_copy(sbufs[b], ni_hbm.at[pl.ds(off, _C)],
                                  wsems[b]).wait()
            pltpu.make_async_copy(dbufs[b], nj_hbm.at[pl.ds(off, _C)],
                                  wsems[b]).wait()

    scatter_scratch = (
        [pltpu.VMEM((1, _C), jnp.int32)] * _NBUF_S
        + [pltpu.VMEM((_C, _D), jnp.float32)] * _NBUF_S
        + [pltpu.SemaphoreType.DMA] * (2 * _NBUF_S)
        + [pltpu.VMEM_SHARED((_N_NODES, _D), jnp.float32)]
    )

    @functools.partial(
        pl.kernel,
        out_type=jax.ShapeDtypeStruct((_NC, _N_NODES, _D), jnp.float32),
        mesh=mesh,
        scratch_types=scatter_scratch,
    )
    def sc_scatter(z_hbm, src4_hbm, zeros_hbm, upd_hbm, *scr):
        ibufs = scr[0:_NBUF_S]
        zbufs = scr[_NBUF_S:2 * _NBUF_S]
        lsems = scr[2 * _NBUF_S:3 * _NBUF_S]
        ssems = scr[3 * _NBUF_S:4 * _NBUF_S]
        accum = scr[4 * _NBUF_S]

        cid = lax.axis_index("c")
        sid = lax.axis_index("s")
        wid = cid * _NS + sid
        base = wid * _EPW

        pltpu.sync_copy(zeros_hbm.at[pl.ds(sid * _RPT, _RPT)],
                        accum.at[pl.ds(sid * _RPT, _RPT)])

        @pl.when(sid == 0)
        def _():
            pltpu.sync_copy(zeros_hbm.at[pl.ds(_NS * _RPT, _RTAIL)],
                            accum.at[pl.ds(_NS * _RPT, _RTAIL)])

        plsc.subcore_barrier()

        def start_loads(j, b):
            off = base + j * _C
            pltpu.async_copy(src4_hbm.at[wid, j], ibufs[b], lsems[b])
            pltpu.async_copy(z_hbm.at[pl.ds(off, _C)], zbufs[b], lsems[b])

        def wait_loads(j, b):
            off = base + j * _C
            pltpu.make_async_copy(src4_hbm.at[wid, j], ibufs[b],
                                  lsems[b]).wait()
            pltpu.make_async_copy(z_hbm.at[pl.ds(off, _C)], zbufs[b],
                                  lsems[b]).wait()

        for b in range(_NBUF_S):
            start_loads(b, b)

        def group(g, carry):
            for b in range(_NBUF_S):
                j = g * _NBUF_S + b
                wait_loads(j, b)
                pltpu.async_copy(zbufs[b], accum.at[ibufs[b].at[0]], ssems[b],
                                 add=True)

            @pl.when(g < _NGRP_S - 1)
            def _():
                for b in range(_NBUF_S):
                    j = g * _NBUF_S + b
                    pltpu.make_async_copy(zbufs[b], accum.at[ibufs[b].at[0]],
                                          ssems[b]).wait()
                    start_loads(j + _NBUF_S, b)
            return carry

        lax.fori_loop(0, _NGRP_S, group, 0)

        for b in range(_NBUF_S):
            pltpu.make_async_copy(zbufs[b], accum.at[ibufs[b].at[0]],
                                  ssems[b]).wait()
        jt = _NGRP_S * _NBUF_S
        start_loads(jt, 0)
        wait_loads(jt, 0)
        pltpu.async_copy(zbufs[0], accum.at[ibufs[0].at[0]], ssems[0],
                         add=True)
        pltpu.make_async_copy(zbufs[0], accum.at[ibufs[0].at[0]],
                              ssems[0]).wait()

        plsc.subcore_barrier()

        pltpu.sync_copy(accum.at[pl.ds(sid * _RPT, _RPT)],
                        upd_hbm.at[cid, pl.ds(sid * _RPT, _RPT)])

        @pl.when(sid == 0)
        def _():
            pltpu.sync_copy(accum.at[pl.ds(_NS * _RPT, _RTAIL)],
                            upd_hbm.at[cid, pl.ds(_NS * _RPT, _RTAIL)])

    return sc_gather, sc_scatter


def _embed_body(n_ref, w_ref, o_ref):
    o_ref[...] = _sigmoid(_dot(n_ref[...], w_ref[...]))


def _embed(nodes, emb_w):
    return pl.pallas_call(
        _embed_body,
        grid=(_NODE_GRID,),
        in_specs=[
            pl.BlockSpec((_NODE_TILE, _D), lambda i: (i, 0)),
            pl.BlockSpec((_D, _D), lambda i: (0, 0)),
        ],
        out_specs=pl.BlockSpec((_NODE_TILE, _D), lambda i: (i, 0)),
        out_shape=jax.ShapeDtypeStruct((_N_NODES, _D), jnp.float32),
    )(nodes, emb_w)


def _edge_body(ni_ref, nj_ref, rij_ref, cs_ref, pw_ref,
               g1, g2, g3, m1, m2, m3, w1, w2, w2g, z_ref):
    ii = ni_ref[...]
    jj = nj_ref[...]
    inv_r = 1.0 / rij_ref[...]
    dd = (jj - ii) * inv_r
    gate = _sigmoid(_dot(ii, g1[...]) + _dot(jj, g2[...]) + _dot(dd, g3[...]))
    mlp = _elu(_dot(ii, m1[...]) + _dot(jj, m2[...]) + _dot(dd, m3[...]))
    pw = pw_ref[...]
    zc = _dot(cs_ref[...], w1[...]) + _dot(pw, w2[...]) * _sigmoid(_dot(pw, w2g[...]))
    z_ref[...] = gate * mlp * zc


def _edge(ni, nj, rij2, cs, pw, g1, g2, g3, m1, m2, m3, w1, w2, w2g):
    ew = lambda i: (i, 0)
    cw = lambda i: (0, 0)
    return pl.pallas_call(
        _edge_body,
        grid=(_EDGE_GRID,),
        in_specs=[
            pl.BlockSpec((_EDGE_TILE, _D), ew),
            pl.BlockSpec((_EDGE_TILE, _D), ew),
            pl.BlockSpec((_EDGE_TILE, 1), ew),
            pl.BlockSpec((_EDGE_TILE, 36), ew),
            pl.BlockSpec((_EDGE_TILE, 26), ew),
            pl.BlockSpec((_D, _D), cw),
            pl.BlockSpec((_D, _D), cw),
            pl.BlockSpec((_D, _D), cw),
            pl.BlockSpec((_D, _D), cw),
            pl.BlockSpec((_D, _D), cw),
            pl.BlockSpec((_D, _D), cw),
            pl.BlockSpec((36, _D), cw),
            pl.BlockSpec((26, _D), cw),
            pl.BlockSpec((26, _D), cw),
        ],
        out_specs=pl.BlockSpec((_EDGE_TILE, _D), ew),
        out_shape=jax.ShapeDtypeStruct((_N_EDGES, _D), jnp.float32),
    )(ni, nj, rij2, cs, pw, g1, g2, g3, m1, m2, m3, w1, w2, w2g)


def _node_body(x_ref, u_ref, q_ref, psi, p1, p2, xo_ref, qo_ref):
    xv = x_ref[...] + u_ref[0] + u_ref[1]
    xn = _elu(_dot(xv, psi[...]))
    zp = _elu(_dot(xn, p1[...]))
    xo_ref[...] = xn
    qo_ref[...] = q_ref[...] + zp * _dot(xn, p2[...])


def _node(x, upd, q, psi, p1, p2):
    nw = lambda i: (i, 0)
    cw = lambda i: (0, 0)
    return pl.pallas_call(
        _node_body,
        grid=(_NODE_GRID,),
        in_specs=[
            pl.BlockSpec((_NODE_TILE, _D), nw),
            pl.BlockSpec((_NC, _NODE_TILE, _D), lambda i: (0, i, 0)),
            pl.BlockSpec((_NODE_TILE, _D), nw),
            pl.BlockSpec((_D, _D), cw),
            pl.BlockSpec((_D, _D), cw),
            pl.BlockSpec((_D, _D), cw),
        ],
        out_specs=[
            pl.BlockSpec((_NODE_TILE, _D), nw),
            pl.BlockSpec((_NODE_TILE, _D), nw),
        ],
        out_shape=[
            jax.ShapeDtypeStruct((_N_NODES, _D), jnp.float32),
            jax.ShapeDtypeStruct((_N_NODES, _D), jnp.float32),
        ],
    )(x, upd, q, psi, p1, p2)


def _final_body(q_ref, gi_ref, w1, w2, w3, y_ref, pooled):
    i = pl.program_id(0)

    @pl.when(i == 0)
    def _():
        pooled[...] = jnp.zeros_like(pooled)

    gi = gi_ref[0, 0, :]
    gids = lax.broadcasted_iota(jnp.int32, (_NGRAPH, _NODE_TILE), 0)
    onehot = (gids == gi[None, :]).astype(jnp.float32)
    pooled[...] += _dot(onehot, q_ref[...])

    @pl.when(i == _NODE_GRID - 1)
    def _():
        y = _elu(_dot(pooled[...], w1[...]))
        y = _elu(_dot(y, w2[...]))
        y_ref[...] = _dot(y, w3[...])


def _final(q, gi3, w1, w2, w3):
    cw = lambda i: (0, 0)
    return pl.pallas_call(
        _final_body,
        grid=(_NODE_GRID,),
        in_specs=[
            pl.BlockSpec((_NODE_TILE, _D), lambda i: (i, 0)),
            pl.BlockSpec((1, 1, _NODE_TILE), lambda i: (i, 0, 0)),
            pl.BlockSpec((_D, 64), cw),
            pl.BlockSpec((64, 42), cw),
            pl.BlockSpec((42, 1), cw),
        ],
        out_specs=pl.BlockSpec((_NGRAPH, 1), cw),
        out_shape=jax.ShapeDtypeStruct((_NGRAPH, 1), jnp.float32),
        scratch_shapes=[pltpu.VMEM((_NGRAPH, _D), jnp.float32)],
    )(q, gi3, w1, w2, w3)


@jax.jit
def _run(nodes, src, dst, rij2, cs, pw, gi3, params):
    sc_gather, sc_scatter = _sc_kernels()
    src4 = src.reshape(_NW, _NCHUNK, 1, _C)
    zeros_nodes = jnp.zeros((_N_NODES, _D), jnp.float32)
    x = _embed(nodes, params["emb_W"])
    q = jnp.zeros((_N_NODES, _D), jnp.float32)
    for bp in params["blocks"]:
        g1, g2, g3 = jnp.split(bp["lin_gate"], 3, axis=0)
        m1, m2, m3 = jnp.split(bp["lin_mlp"], 3, axis=0)
        ni, nj = sc_gather(x, src, dst)
        z = _edge(ni, nj, rij2, cs, pw, g1, g2, g3, m1, m2, m3,
                  bp["lin1_vec"], bp["lin2_vec"], bp["lin2_vec_gate"])
        upd = sc_scatter(z, src4, zeros_nodes)
        x, q = _node(x, upd, q, bp["psi_W"], bp["pool_W1"], bp["pool_W2"])
    return _final(q, gi3, params["lr"][0], params["lr"][1], params["lr"][2])


def kernel(nodes, edge_sources, edge_targets, rij, combine_sets, plane_wave,
           graph_indices, node_counts, params):
    del node_counts
    src = edge_sources.astype(jnp.int32)
    dst = edge_targets.astype(jnp.int32)
    rij2 = rij.reshape(_N_EDGES, 1)
    gi3 = graph_indices.astype(jnp.int32).reshape(_NODE_GRID, 1, _NODE_TILE)
    return _run(nodes, src, dst, rij2, combine_sets, plane_wave, gi3, params)
